# baseline (speedup 1.0000x reference)
import numpy as np
import jax
import jax.numpy as jnp
from jax import lax
from jax.experimental import pallas as pl
from jax.experimental.pallas import tpu as pltpu
from jax.experimental.pallas import tpu_sc as plsc

_VOCAB = 100000
_DIM = 1024
_MAX_LEN = 2048
_B = 4
_S = 2048

_NC = 2
_NS = 16
_NW = _NC * _NS
_TOK = _B * _S
_PPW = _S // _NW
_CH = 16
_QPB = _PPW // _CH
_NCH = _B * _QPB
_L = 16
_VPR = _DIM // _L


def _pe_table() -> np.ndarray:
    pos = np.arange(_MAX_LEN, dtype=np.float32)[:, None]
    i = np.arange(_DIM, dtype=np.float32)[None, :]
    angle_rates = 1.0 / np.power(10000.0, (2.0 * np.floor(i / 2.0)) / _DIM)
    angles = pos * angle_rates
    pe = np.zeros((_MAX_LEN, _DIM), dtype=np.float32)
    pe[:, 0::2] = np.sin(angles[:, 0::2])
    pe[:, 1::2] = np.cos(angles[:, 1::2])
    return pe


_PE = _pe_table()


_NBUF = 3


def _embed_body(x_hbm, tab_hbm, pe_hbm, out_hbm,
                idx_v, pe_buf, buf0, buf1, buf2,
                sem_pe, sem_i, sem_g0, sem_g1, sem_g2,
                sem_s0, sem_s1, sem_s2):
    wid = lax.axis_index("s") * _NC + lax.axis_index("c")
    pbase = wid * _PPW

    pe_cp = pltpu.async_copy(pe_hbm.at[pl.ds(pbase, _PPW), :], pe_buf, sem_pe)
    idx_cps = [
        pltpu.async_copy(x_hbm.at[pl.ds(b * _S + pbase, _PPW)],
                         idx_v.at[pl.ds(b * _PPW, _PPW)], sem_i)
        for b in range(_B)
    ]
    for cp in idx_cps:
        cp.wait()

    bufs = (buf0, buf1, buf2)
    gsems = (sem_g0, sem_g1, sem_g2)
    ssems = (sem_s0, sem_s1, sem_s2)

    def tok_base(c):
        b, q = c // _QPB, c % _QPB
        return b * _S + pbase + q * _CH

    def gather(c):
        ioff = (c // _QPB) * _PPW + (c % _QPB) * _CH
        p = c % _NBUF
        return pltpu.async_copy(
            tab_hbm.at[idx_v.at[pl.ds(ioff, _CH)]], bufs[p], gsems[p])

    gathers = [gather(0), gather(1), None]
    pe_cp.wait()
    stores = [None, None, None]
    for c in range(_NCH):
        p = c % _NBUF
        gathers[p].wait()
        if c + 2 < _NCH:
            p2 = (c + 2) % _NBUF
            if stores[p2] is not None:
                stores[p2].wait()
            gathers[p2] = gather(c + 2)

        q = c % _QPB

        def _row(r, carry):
            buf = bufs[p]
            for j in range(_VPR):
                v = pe_buf[q * _CH + r, pl.ds(j * _L, _L)]
                plsc.addupdate(buf.at[r, pl.ds(j * _L, _L)], v)
            return carry

        lax.fori_loop(0, _CH, _row, 0)

        stores[p] = pltpu.async_copy(
            bufs[p], out_hbm.at[pl.ds(tok_base(c), _CH), :], ssems[p])
    for st in stores:
        if st is not None:
            st.wait()


_embed = pl.kernel(
    _embed_body,
    out_type=jax.ShapeDtypeStruct((_TOK, _DIM), jnp.float32),
    mesh=plsc.VectorSubcoreMesh(core_axis_name="c", subcore_axis_name="s"),
    scratch_types=[
        pltpu.VMEM((_B * _PPW,), jnp.int32),
        pltpu.VMEM((_PPW, _DIM), jnp.float32),
        pltpu.VMEM((_CH, _DIM), jnp.float32),
        pltpu.VMEM((_CH, _DIM), jnp.float32),
        pltpu.VMEM((_CH, _DIM), jnp.float32),
        pltpu.SemaphoreType.DMA,
        pltpu.SemaphoreType.DMA,
        pltpu.SemaphoreType.DMA,
        pltpu.SemaphoreType.DMA,
        pltpu.SemaphoreType.DMA,
        pltpu.SemaphoreType.DMA,
        pltpu.SemaphoreType.DMA,
        pltpu.SemaphoreType.DMA,
    ],
)


@jax.jit
def kernel(x, token_table):
    pe = jnp.asarray(_PE)
    out = _embed(x.reshape(_TOK), token_table, pe)
    return out.reshape(_B, _S, _DIM)


# speedup vs baseline: 1.1768x; 1.1768x over previous
import numpy as np
import jax
import jax.numpy as jnp
from jax import lax
from jax.experimental import pallas as pl
from jax.experimental.pallas import tpu as pltpu
from jax.experimental.pallas import tpu_sc as plsc

_VOCAB = 100000
_DIM = 1024
_MAX_LEN = 2048
_B = 4
_S = 2048

_NC = 2
_NS = 16
_NW = _NC * _NS
_TOK = _B * _S
_PPW = _S // _NW
_CH = 16
_QPB = _PPW // _CH
_NCH = _B * _QPB
_L = 16
_VPR = _DIM // _L


def _pe_table() -> np.ndarray:
    pos = np.arange(_MAX_LEN, dtype=np.float32)[:, None]
    i = np.arange(_DIM, dtype=np.float32)[None, :]
    angle_rates = 1.0 / np.power(10000.0, (2.0 * np.floor(i / 2.0)) / _DIM)
    angles = pos * angle_rates
    pe = np.zeros((_MAX_LEN, _DIM), dtype=np.float32)
    pe[:, 0::2] = np.sin(angles[:, 0::2])
    pe[:, 1::2] = np.cos(angles[:, 1::2])
    return pe


_PE = _pe_table()


def _embed_body(x_hbm, tab_hbm, pe_hbm, out_hbm,
                idx_v, pe_buf, buf0, buf1,
                sem_pe, sem_i, sem_g0, sem_g1, sem_s0, sem_s1):
    wid = lax.axis_index("s") * _NC + lax.axis_index("c")
    pbase = wid * _PPW

    pe_cp = pltpu.async_copy(pe_hbm.at[pl.ds(pbase, _PPW), :], pe_buf, sem_pe)
    idx_cps = [
        pltpu.async_copy(x_hbm.at[pl.ds(b * _S + pbase, _PPW)],
                         idx_v.at[pl.ds(b * _PPW, _PPW)], sem_i)
        for b in range(_B)
    ]
    for cp in idx_cps:
        cp.wait()

    bufs = (buf0, buf1)
    gsems = (sem_g0, sem_g1)
    ssems = (sem_s0, sem_s1)

    def ioff(c):
        return (c // _QPB) * _PPW + (c % _QPB) * _CH

    def tok_base(c):
        return (c // _QPB) * _S + pbase + (c % _QPB) * _CH

    def gather(c, k):
        return pltpu.async_copy(
            tab_hbm.at[idx_v.at[pl.ds(ioff(c), _CH)]], bufs[k], gsems[k])

    gather(0, 0)
    gather(1, 1)
    pe_cp.wait()

    def _round(r, carry):
        c0 = 2 * r
        for k in range(2):
            c = c0 + k
            buf, gsem, ssem = bufs[k], gsems[k], ssems[k]
            pltpu.make_async_copy(
                tab_hbm.at[idx_v.at[pl.ds(ioff(c), _CH)]], buf, gsem).wait()

            q = c % _QPB

            @plsc.parallel_loop(0, _CH, step=1)
            def _row(rr):
                for j in range(_VPR):
                    v = pe_buf[q * _CH + rr, pl.ds(j * _L, _L)]
                    plsc.addupdate(buf.at[rr, pl.ds(j * _L, _L)], v)

            st = pltpu.async_copy(
                buf, out_hbm.at[pl.ds(tok_base(c), _CH), :], ssem)

            @pl.when(c + 2 < _NCH)
            def _():
                st.wait()
                gather(c + 2, k)
        return carry

    lax.fori_loop(0, _NCH // 2, _round, 0)
    for k in range(2):
        pltpu.make_async_copy(
            bufs[k], out_hbm.at[pl.ds(tok_base(_NCH - 2 + k), _CH), :],
            ssems[k]).wait()


_embed = pl.kernel(
    _embed_body,
    out_type=jax.ShapeDtypeStruct((_TOK, _DIM), jnp.float32),
    mesh=plsc.VectorSubcoreMesh(core_axis_name="c", subcore_axis_name="s"),
    scratch_types=[
        pltpu.VMEM((_B * _PPW,), jnp.int32),
        pltpu.VMEM((_PPW, _DIM), jnp.float32),
        pltpu.VMEM((_CH, _DIM), jnp.float32),
        pltpu.VMEM((_CH, _DIM), jnp.float32),
        pltpu.SemaphoreType.DMA,
        pltpu.SemaphoreType.DMA,
        pltpu.SemaphoreType.DMA,
        pltpu.SemaphoreType.DMA,
        pltpu.SemaphoreType.DMA,
        pltpu.SemaphoreType.DMA,
    ],
)


@jax.jit
def kernel(x, token_table):
    pe = jnp.asarray(_PE)
    out = _embed(x.reshape(_TOK), token_table, pe)
    return out.reshape(_B, _S, _DIM)


# speedup vs baseline: 1.1978x; 1.0178x over previous
import numpy as np
import jax
import jax.numpy as jnp
from jax import lax
from jax.experimental import pallas as pl
from jax.experimental.pallas import tpu as pltpu
from jax.experimental.pallas import tpu_sc as plsc

_VOCAB = 100000
_DIM = 1024
_MAX_LEN = 2048
_B = 4
_S = 2048

_NC = 2
_NS = 16
_NW = _NC * _NS
_PPW = _S // _NW
_CH = 16
_QPB = _PPW // _CH
_NCH = _B * _QPB
_L = 16
_VPR = _DIM // _L


def _pe_table() -> np.ndarray:
    pos = np.arange(_MAX_LEN, dtype=np.float32)[:, None]
    i = np.arange(_DIM, dtype=np.float32)[None, :]
    angle_rates = 1.0 / np.power(10000.0, (2.0 * np.floor(i / 2.0)) / _DIM)
    angles = pos * angle_rates
    pe = np.zeros((_MAX_LEN, _DIM), dtype=np.float32)
    pe[:, 0::2] = np.sin(angles[:, 0::2])
    pe[:, 1::2] = np.cos(angles[:, 1::2])
    return pe


_PE_FLAT = _pe_table().reshape(-1)


def _embed_body(x_hbm, tab_hbm, pe_hbm, out_hbm,
                idx_v, pe_buf, buf0, buf1,
                sem_pe, sem_i, sem_g0, sem_g1, sem_s0, sem_s1):
    wid = lax.axis_index("s") * _NC + lax.axis_index("c")
    pbase = wid * _PPW

    pe_cp = pltpu.async_copy(
        pe_hbm.at[pl.ds(pbase * _DIM, _PPW * _DIM)], pe_buf, sem_pe)
    idx_cps = [
        pltpu.async_copy(x_hbm.at[b, pl.ds(pbase, _PPW)],
                         idx_v.at[pl.ds(b * _PPW, _PPW)], sem_i)
        for b in range(_B)
    ]
    for cp in idx_cps:
        cp.wait()

    bufs = (buf0, buf1)
    gsems = (sem_g0, sem_g1)
    ssems = (sem_s0, sem_s1)

    def ioff(c):
        return (c // _QPB) * _PPW + (c % _QPB) * _CH

    def out_slice(c):
        return out_hbm.at[c // _QPB,
                          pl.ds(pbase + (c % _QPB) * _CH, _CH), :]

    def gather(c, k):
        return pltpu.async_copy(
            tab_hbm.at[idx_v.at[pl.ds(ioff(c), _CH)]], bufs[k], gsems[k])

    gather(0, 0)
    gather(1, 1)
    pe_cp.wait()

    def _round(r, carry):
        c0 = 2 * r
        for k in range(2):
            c = c0 + k
            buf, gsem, ssem = bufs[k], gsems[k], ssems[k]
            pltpu.make_async_copy(
                tab_hbm.at[idx_v.at[pl.ds(ioff(c), _CH)]], buf, gsem).wait()

            q = c % _QPB

            @plsc.parallel_loop(0, _CH, step=1)
            def _row(rr):
                rbase = (q * _CH + rr) * _DIM
                for j in range(_VPR):
                    v = pe_buf[pl.ds(rbase + j * _L, _L)]
                    plsc.addupdate(buf.at[rr, pl.ds(j * _L, _L)], v)

            st = pltpu.async_copy(buf, out_slice(c), ssem)

            @pl.when(c + 2 < _NCH)
            def _():
                st.wait()
                gather(c + 2, k)
        return carry

    lax.fori_loop(0, _NCH // 2, _round, 0)
    for k in range(2):
        pltpu.make_async_copy(bufs[k], out_slice(_NCH - 2 + k),
                              ssems[k]).wait()


_embed = pl.kernel(
    _embed_body,
    out_type=jax.ShapeDtypeStruct((_B, _S, _DIM), jnp.float32),
    mesh=plsc.VectorSubcoreMesh(core_axis_name="c", subcore_axis_name="s"),
    scratch_types=[
        pltpu.VMEM((_B * _PPW,), jnp.int32),
        pltpu.VMEM((_PPW * _DIM,), jnp.float32),
        pltpu.VMEM((_CH, _DIM), jnp.float32),
        pltpu.VMEM((_CH, _DIM), jnp.float32),
        pltpu.SemaphoreType.DMA,
        pltpu.SemaphoreType.DMA,
        pltpu.SemaphoreType.DMA,
        pltpu.SemaphoreType.DMA,
        pltpu.SemaphoreType.DMA,
        pltpu.SemaphoreType.DMA,
    ],
)


@jax.jit
def kernel(x, token_table):
    pe = jnp.asarray(_PE_FLAT)
    return _embed(x, token_table, pe)
